# baseline (speedup 1.0000x reference)
import functools

import jax
import jax.numpy as jnp
from jax import lax
from jax.experimental import pallas as pl
from jax.experimental.pallas import tpu as pltpu
from jax.experimental.pallas import tpu_sc as plsc

VOCAB = 100000
BATCH = 128
TOPK = 5
L = 16
NC = 2
NS = 16
NW = NC * NS
NWAVE = 4
WROWS = BATCH // NWAVE
RPW = WROWS // NW
NEG = float(jnp.finfo(jnp.float32).min)
IMAX = int(jnp.iinfo(jnp.int32).max)

CHK = 512
NCHK = 196
TAIL0 = 99968
NTAIL = VOCAB - TAIL0

TCR = 32
CPS = 7168
NCB = 7
CPB = CPS // CHK
HCHK = NCB * CPB
CMW = NCB * 128

CAP = 1024
CBUF = CAP + CHK + L
NRST = CBUF // L


def _chunk_maxes(xm):
    cms = []
    for k in range(CPB):
        cms.append(jnp.max(xm[:, k * CHK:(k + 1) * CHK], axis=1,
                           keepdims=True))
    cms.append(jnp.full((TCR, 128 - CPB), NEG, jnp.float32))
    return jnp.concatenate(cms, axis=1)


def _lane_sums(ex):
    acc = ex[:, 0:128]
    for g in range(1, CPS // 128):
        acc = acc + ex[:, g * 128:(g + 1) * 128]
    return acc


def _tc_body(xa_ref, xb_ref, cma_ref, cmb_ref, den_ref, tail_ref):
    c = pl.program_id(1)
    xa = xa_ref[...]
    xb = xb_ref[...]
    jio = lax.broadcasted_iota(jnp.int32, (TCR, CPS), 1)
    xbm = jnp.where(jio < VOCAB - (c + NCB) * CPS, xb, NEG)

    acc = _lane_sums(jnp.exp(xa)) + _lane_sums(jnp.exp(xbm))

    @pl.when(c == 0)
    def _():
        den_ref[...] = jnp.zeros_like(den_ref)

    den_ref[...] = den_ref[...] + acc

    cma_ref[...] = _chunk_maxes(xa)
    cmb_ref[...] = _chunk_maxes(xbm)

    @pl.when(c == NCB - 1)
    def _():
        lo = TAIL0 - (2 * NCB - 1) * CPS
        tail_ref[...] = xb[:, lo:lo + NTAIL]


def _tc_summarize(logits, w):
    roff = w * (WROWS // TCR)
    return pl.pallas_call(
        _tc_body,
        grid=(WROWS // TCR, NCB),
        in_specs=[
            pl.BlockSpec((TCR, CPS), lambda i, c: (i + roff, c)),
            pl.BlockSpec((TCR, CPS), lambda i, c: (i + roff, c + NCB)),
        ],
        out_specs=[
            pl.BlockSpec((TCR, 128), lambda i, c: (i, c)),
            pl.BlockSpec((TCR, 128), lambda i, c: (i, c)),
            pl.BlockSpec((TCR, 128), lambda i, c: (i, 0)),
            pl.BlockSpec((TCR, NTAIL), lambda i, c: (i, 0)),
        ],
        out_shape=[
            jax.ShapeDtypeStruct((WROWS, CMW), jnp.float32),
            jax.ShapeDtypeStruct((WROWS, CMW), jnp.float32),
            jax.ShapeDtypeStruct((WROWS, 128), jnp.float32),
            jax.ShapeDtypeStruct((WROWS, NTAIL), jnp.float32),
        ],
    )(logits, logits)


def _sc_body(woff, logits_hbm, cma_hbm, cmb_hbm, den_hbm, tail_hbm, g_hbm,
             probs_hbm, ints_hbm,
             cma_v, cmb_v, den_v, tail_v, chunk_s, cval_v, cidx_v, cchk_v,
             g_v, stp_v, sti_v):
    wid = lax.axis_index("s") * NC + lax.axis_index("c")
    iota = lax.broadcasted_iota(jnp.int32, (L,), 0)

    def do_row(rr, prev_used):
        rl = wid * RPW + rr
        r = rl + woff
        pltpu.sync_copy(cma_hbm.at[rl], cma_v)
        pltpu.sync_copy(cmb_hbm.at[rl], cmb_v)
        pltpu.sync_copy(den_hbm.at[rl], den_v)
        pltpu.sync_copy(tail_hbm.at[rl], tail_v)
        pltpu.sync_copy(g_hbm.at[rl], g_v)

        dacc = den_v[pl.ds(0, L)]
        for j in range(1, 128 // L):
            dacc = dacc + den_v[pl.ds(j * L, L)]
        denom = jnp.sum(dacc)

        cms = ([cma_v[pl.ds(j * 128, L)] for j in range(NCB)]
               + [cmb_v[pl.ds(j * 128, L)] for j in range(NCB)])
        bases = ([j * CPB for j in range(NCB)]
                 + [HCHK + j * CPB for j in range(NCB)])
        cur = cms
        thr = jnp.float32(0)
        for _k in range(TOPK):
            thr = jnp.max(functools.reduce(jnp.maximum, cur))
            cur = [jnp.where(c == thr, NEG, c) for c in cur]
        thr_v = jnp.full((L,), thr, jnp.float32)

        ptr = jnp.int32(0)
        for j in range(2 * NCB):
            m = cms[j] >= thr_v
            plsc.store_compressed(cchk_v.at[pl.ds(ptr, L)], iota + bases[j],
                                  mask=m)
            ptr = ptr + jnp.sum(m.astype(jnp.int32))
        ncand = ptr

        def rst(cc, _):
            cval_v[pl.ds(cc * L, L)] = jnp.full((L,), NEG, jnp.float32)
            cidx_v[pl.ds(cc * L, L)] = jnp.full((L,), IMAX, jnp.int32)
            return 0

        lax.fori_loop(0, prev_used, rst, 0)

        def cand_body(i, p):
            jv = i // L
            lane = i - jv * L
            idsv = cchk_v[pl.ds(jv * L, L)]
            cid = jnp.max(jnp.where(iota == lane, idsv, -1))
            off = pl.multiple_of(cid * CHK, 128)

            def f_full(_):
                pltpu.sync_copy(logits_hbm.at[r].at[pl.ds(off, CHK)],
                                chunk_s)
                return 0

            def f_short(_):
                pltpu.sync_copy(logits_hbm.at[r].at[pl.ds(off, 128)],
                                chunk_s.at[pl.ds(0, 128)])
                return 0

            lax.cond(cid < NCHK - 1, f_full, f_short, 0)

            nvalid = jnp.where(cid < NCHK - 1, CHK, TAIL0 - off)
            for k in range(CHK // L):
                x = chunk_s[pl.ds(k * L, L)]
                gidx = iota + (off + k * L)
                m = (x >= thr_v) & (gidx < off + nvalid)
                plsc.store_compressed(cval_v.at[pl.ds(p, L)], x, mask=m)
                plsc.store_compressed(cidx_v.at[pl.ds(p, L)], gidx, mask=m)
                p = p + jnp.sum(m.astype(jnp.int32))
            return jnp.minimum(p, CAP)

        cp = lax.fori_loop(0, ncand, cand_body, jnp.int32(0))

        for t in range(NTAIL // L):
            x = tail_v[pl.ds(t * L, L)]
            gidx = iota + (TAIL0 + t * L)
            m = x >= thr_v
            plsc.store_compressed(cval_v.at[pl.ds(cp, L)], x, mask=m)
            plsc.store_compressed(cidx_v.at[pl.ds(cp, L)], gidx, mask=m)
            cp = cp + jnp.sum(m.astype(jnp.int32))

        ptrf = jnp.minimum(cp, CAP + NTAIL)
        nv = (ptrf + (L - 1)) // L

        topv = jnp.full((L,), NEG, jnp.float32)
        topi = jnp.full((L,), IMAX, jnp.int32)
        pv = jnp.float32(jnp.finfo(jnp.float32).max)
        pi = jnp.int32(-1)
        for k in range(TOPK):
            def sel_body(c, carry, pv=pv, pi=pi):
                bv, bi = carry
                v = cval_v[pl.ds(c * L, L)]
                i = cidx_v[pl.ds(c * L, L)]
                elig = (v < pv) | ((v == pv) & (i > pi))
                better = elig & ((v > bv) | ((v == bv) & (i < bi)))
                return (jnp.where(better, v, bv), jnp.where(better, i, bi))

            bv, bi = lax.fori_loop(
                0, nv, sel_body,
                (jnp.full((L,), NEG, jnp.float32),
                 jnp.full((L,), IMAX, jnp.int32)))
            mv = jnp.max(bv)
            mi = jnp.min(jnp.where(bv == mv, bi, IMAX))
            topv = jnp.where(iota == k, mv, topv)
            topi = jnp.where(iota == k, mi, topi)
            pv, pi = mv, mi

        p_out = jnp.where(iota < TOPK, jnp.exp(topv) / denom, 0.0)
        gv = g_v[...]
        score = jnp.where(iota < TOPK, topv + gv, NEG)
        ms = jnp.max(score)
        ix = jnp.min(jnp.where(score == ms, iota, L))
        xv = jnp.max(jnp.where(iota == ix, topi, -1))

        stp_v[...] = p_out
        sti_v[...] = jnp.where(iota == TOPK, xv, topi)
        pltpu.sync_copy(stp_v, probs_hbm.at[rl])
        pltpu.sync_copy(sti_v, ints_hbm.at[rl])
        return jnp.minimum((ptrf + 2 * L - 1) // L + 1, NRST)

    lax.fori_loop(0, RPW, do_row, jnp.int32(NRST))


def _sc_topk_sample(logits, cma, cmb, den, tail, gpad, woff):
    mesh = plsc.VectorSubcoreMesh(core_axis_name="c", subcore_axis_name="s")
    f = pl.kernel(
        functools.partial(_sc_body, woff),
        out_type=(jax.ShapeDtypeStruct((WROWS, L), jnp.float32),
                  jax.ShapeDtypeStruct((WROWS, L), jnp.int32)),
        mesh=mesh,
        compiler_params=pltpu.CompilerParams(needs_layout_passes=False,
                                             use_tc_tiling_on_sc=True),
        scratch_types=[
            pltpu.VMEM((CMW,), jnp.float32),
            pltpu.VMEM((CMW,), jnp.float32),
            pltpu.VMEM((128,), jnp.float32),
            pltpu.VMEM((NTAIL,), jnp.float32),
            pltpu.VMEM((CHK,), jnp.float32),
            pltpu.VMEM((CBUF,), jnp.float32),
            pltpu.VMEM((CBUF,), jnp.int32),
            pltpu.VMEM((2 * NCB * L + L,), jnp.int32),
            pltpu.VMEM((L,), jnp.float32),
            pltpu.VMEM((L,), jnp.float32),
            pltpu.VMEM((L,), jnp.int32),
        ],
    )
    return f(logits, cma, cmb, den, tail, gpad)


def kernel(logits):
    g = jax.random.gumbel(jax.random.key(42), (BATCH, TOPK), jnp.float32)
    gpad = jnp.zeros((BATCH, L), jnp.float32).at[:, :TOPK].set(g)
    pp, ii = [], []
    for w in range(NWAVE):
        cma, cmb, den, tail = _tc_summarize(logits, w)
        p, i = _sc_topk_sample(logits, cma, cmb, den, tail,
                               gpad[w * WROWS:(w + 1) * WROWS], w * WROWS)
        pp.append(p)
        ii.append(i)
    probs_pad = jnp.concatenate(pp, axis=0)
    ints_pad = jnp.concatenate(ii, axis=0)
    topk_probs = probs_pad[:, :TOPK]
    topk_indices = ints_pad[:, :TOPK]
    xcol = ints_pad[:, TOPK:TOPK + 1]
    return xcol, topk_probs, topk_indices


# speedup vs baseline: 1.0539x; 1.0539x over previous
import functools

import jax
import jax.numpy as jnp
from jax import lax
from jax.experimental import pallas as pl
from jax.experimental.pallas import tpu as pltpu
from jax.experimental.pallas import tpu_sc as plsc

VOCAB = 100000
BATCH = 128
TOPK = 5
L = 16
NC = 2
NS = 16
NW = NC * NS
NWAVE = 2
WROWS = BATCH // NWAVE
RPW = WROWS // NW
NEG = float(jnp.finfo(jnp.float32).min)
IMAX = int(jnp.iinfo(jnp.int32).max)

CHK = 512
NCHK = 196
TAIL0 = 99968
NTAIL = VOCAB - TAIL0

TCR = 32
CPS = 7168
NCB = 7
CPB = CPS // CHK
HCHK = NCB * CPB
CMW = NCB * 128

CAP = 1024
CBUF = CAP + CHK + L
NRST = CBUF // L


def _chunk_maxes(xm):
    cms = []
    for k in range(CPB):
        cms.append(jnp.max(xm[:, k * CHK:(k + 1) * CHK], axis=1,
                           keepdims=True))
    cms.append(jnp.full((TCR, 128 - CPB), NEG, jnp.float32))
    return jnp.concatenate(cms, axis=1)


def _lane_sums(ex):
    acc = ex[:, 0:128]
    for g in range(1, CPS // 128):
        acc = acc + ex[:, g * 128:(g + 1) * 128]
    return acc


def _tc_body(xa_ref, xb_ref, cma_ref, cmb_ref, den_ref, tail_ref):
    c = pl.program_id(1)
    xa = xa_ref[...]
    xb = xb_ref[...]
    jio = lax.broadcasted_iota(jnp.int32, (TCR, CPS), 1)
    xbm = jnp.where(jio < VOCAB - (c + NCB) * CPS, xb, NEG)

    acc = _lane_sums(jnp.exp(xa)) + _lane_sums(jnp.exp(xbm))

    @pl.when(c == 0)
    def _():
        den_ref[...] = jnp.zeros_like(den_ref)

    den_ref[...] = den_ref[...] + acc

    cma_ref[...] = _chunk_maxes(xa)
    cmb_ref[...] = _chunk_maxes(xbm)

    @pl.when(c == NCB - 1)
    def _():
        lo = TAIL0 - (2 * NCB - 1) * CPS
        tail_ref[...] = xb[:, lo:lo + NTAIL]


def _tc_summarize(logits, w):
    roff = w * (WROWS // TCR)
    return pl.pallas_call(
        _tc_body,
        grid=(WROWS // TCR, NCB),
        in_specs=[
            pl.BlockSpec((TCR, CPS), lambda i, c: (i + roff, c)),
            pl.BlockSpec((TCR, CPS), lambda i, c: (i + roff, c + NCB)),
        ],
        out_specs=[
            pl.BlockSpec((TCR, 128), lambda i, c: (i, c)),
            pl.BlockSpec((TCR, 128), lambda i, c: (i, c)),
            pl.BlockSpec((TCR, 128), lambda i, c: (i, 0)),
            pl.BlockSpec((TCR, NTAIL), lambda i, c: (i, 0)),
        ],
        out_shape=[
            jax.ShapeDtypeStruct((WROWS, CMW), jnp.float32),
            jax.ShapeDtypeStruct((WROWS, CMW), jnp.float32),
            jax.ShapeDtypeStruct((WROWS, 128), jnp.float32),
            jax.ShapeDtypeStruct((WROWS, NTAIL), jnp.float32),
        ],
    )(logits, logits)


def _sc_body(woff, logits_hbm, cma_hbm, cmb_hbm, den_hbm, tail_hbm, g_hbm,
             probs_hbm, ints_hbm,
             cma_v, cmb_v, den_v, tail_v, chunk_s, cval_v, cidx_v, cchk_v,
             g_v, stp_v, sti_v):
    wid = lax.axis_index("s") * NC + lax.axis_index("c")
    iota = lax.broadcasted_iota(jnp.int32, (L,), 0)

    def do_row(rr, prev_used):
        rl = wid * RPW + rr
        r = rl + woff
        pltpu.sync_copy(cma_hbm.at[rl], cma_v)
        pltpu.sync_copy(cmb_hbm.at[rl], cmb_v)
        pltpu.sync_copy(den_hbm.at[rl], den_v)
        pltpu.sync_copy(tail_hbm.at[rl], tail_v)
        pltpu.sync_copy(g_hbm.at[rl], g_v)

        dacc = den_v[pl.ds(0, L)]
        for j in range(1, 128 // L):
            dacc = dacc + den_v[pl.ds(j * L, L)]
        denom = jnp.sum(dacc)

        cms = ([cma_v[pl.ds(j * 128, L)] for j in range(NCB)]
               + [cmb_v[pl.ds(j * 128, L)] for j in range(NCB)])
        bases = ([j * CPB for j in range(NCB)]
                 + [HCHK + j * CPB for j in range(NCB)])
        cur = cms
        thr = jnp.float32(0)
        for _k in range(TOPK):
            thr = jnp.max(functools.reduce(jnp.maximum, cur))
            cur = [jnp.where(c == thr, NEG, c) for c in cur]
        thr_v = jnp.full((L,), thr, jnp.float32)

        ptr = jnp.int32(0)
        for j in range(2 * NCB):
            m = cms[j] >= thr_v
            plsc.store_compressed(cchk_v.at[pl.ds(ptr, L)], iota + bases[j],
                                  mask=m)
            ptr = ptr + jnp.sum(m.astype(jnp.int32))
        ncand = ptr

        def rst(cc, _):
            cval_v[pl.ds(cc * L, L)] = jnp.full((L,), NEG, jnp.float32)
            cidx_v[pl.ds(cc * L, L)] = jnp.full((L,), IMAX, jnp.int32)
            return 0

        lax.fori_loop(0, prev_used, rst, 0)

        def cand_body(i, p):
            jv = i // L
            lane = i - jv * L
            idsv = cchk_v[pl.ds(jv * L, L)]
            cid = jnp.max(jnp.where(iota == lane, idsv, -1))
            off = pl.multiple_of(cid * CHK, 128)

            def f_full(_):
                pltpu.sync_copy(logits_hbm.at[r].at[pl.ds(off, CHK)],
                                chunk_s)
                return 0

            def f_short(_):
                pltpu.sync_copy(logits_hbm.at[r].at[pl.ds(off, 128)],
                                chunk_s.at[pl.ds(0, 128)])
                return 0

            lax.cond(cid < NCHK - 1, f_full, f_short, 0)

            nvalid = jnp.where(cid < NCHK - 1, CHK, TAIL0 - off)
            for k in range(CHK // L):
                x = chunk_s[pl.ds(k * L, L)]
                gidx = iota + (off + k * L)
                m = (x >= thr_v) & (gidx < off + nvalid)
                plsc.store_compressed(cval_v.at[pl.ds(p, L)], x, mask=m)
                plsc.store_compressed(cidx_v.at[pl.ds(p, L)], gidx, mask=m)
                p = p + jnp.sum(m.astype(jnp.int32))
            return jnp.minimum(p, CAP)

        cp = lax.fori_loop(0, ncand, cand_body, jnp.int32(0))

        for t in range(NTAIL // L):
            x = tail_v[pl.ds(t * L, L)]
            gidx = iota + (TAIL0 + t * L)
            m = x >= thr_v
            plsc.store_compressed(cval_v.at[pl.ds(cp, L)], x, mask=m)
            plsc.store_compressed(cidx_v.at[pl.ds(cp, L)], gidx, mask=m)
            cp = cp + jnp.sum(m.astype(jnp.int32))

        ptrf = jnp.minimum(cp, CAP + NTAIL)
        nv = (ptrf + (L - 1)) // L

        topv = jnp.full((L,), NEG, jnp.float32)
        topi = jnp.full((L,), IMAX, jnp.int32)
        pv = jnp.float32(jnp.finfo(jnp.float32).max)
        pi = jnp.int32(-1)
        for k in range(TOPK):
            def sel_body(c, carry, pv=pv, pi=pi):
                bv, bi = carry
                v = cval_v[pl.ds(c * L, L)]
                i = cidx_v[pl.ds(c * L, L)]
                elig = (v < pv) | ((v == pv) & (i > pi))
                better = elig & ((v > bv) | ((v == bv) & (i < bi)))
                return (jnp.where(better, v, bv), jnp.where(better, i, bi))

            bv, bi = lax.fori_loop(
                0, nv, sel_body,
                (jnp.full((L,), NEG, jnp.float32),
                 jnp.full((L,), IMAX, jnp.int32)))
            mv = jnp.max(bv)
            mi = jnp.min(jnp.where(bv == mv, bi, IMAX))
            topv = jnp.where(iota == k, mv, topv)
            topi = jnp.where(iota == k, mi, topi)
            pv, pi = mv, mi

        p_out = jnp.where(iota < TOPK, jnp.exp(topv) / denom, 0.0)
        gv = g_v[...]
        score = jnp.where(iota < TOPK, topv + gv, NEG)
        ms = jnp.max(score)
        ix = jnp.min(jnp.where(score == ms, iota, L))
        xv = jnp.max(jnp.where(iota == ix, topi, -1))

        stp_v[...] = p_out
        sti_v[...] = jnp.where(iota == TOPK, xv, topi)
        pltpu.sync_copy(stp_v, probs_hbm.at[rl])
        pltpu.sync_copy(sti_v, ints_hbm.at[rl])
        return jnp.minimum((ptrf + 2 * L - 1) // L + 1, NRST)

    lax.fori_loop(0, RPW, do_row, jnp.int32(NRST))


def _sc_topk_sample(logits, cma, cmb, den, tail, gpad, woff):
    mesh = plsc.VectorSubcoreMesh(core_axis_name="c", subcore_axis_name="s")
    f = pl.kernel(
        functools.partial(_sc_body, woff),
        out_type=(jax.ShapeDtypeStruct((WROWS, L), jnp.float32),
                  jax.ShapeDtypeStruct((WROWS, L), jnp.int32)),
        mesh=mesh,
        compiler_params=pltpu.CompilerParams(needs_layout_passes=False,
                                             use_tc_tiling_on_sc=True),
        scratch_types=[
            pltpu.VMEM((CMW,), jnp.float32),
            pltpu.VMEM((CMW,), jnp.float32),
            pltpu.VMEM((128,), jnp.float32),
            pltpu.VMEM((NTAIL,), jnp.float32),
            pltpu.VMEM((CHK,), jnp.float32),
            pltpu.VMEM((CBUF,), jnp.float32),
            pltpu.VMEM((CBUF,), jnp.int32),
            pltpu.VMEM((2 * NCB * L + L,), jnp.int32),
            pltpu.VMEM((L,), jnp.float32),
            pltpu.VMEM((L,), jnp.float32),
            pltpu.VMEM((L,), jnp.int32),
        ],
    )
    return f(logits, cma, cmb, den, tail, gpad)


def kernel(logits):
    g = jax.random.gumbel(jax.random.key(42), (BATCH, TOPK), jnp.float32)
    gpad = jnp.zeros((BATCH, L), jnp.float32).at[:, :TOPK].set(g)
    pp, ii = [], []
    for w in range(NWAVE):
        cma, cmb, den, tail = _tc_summarize(logits, w)
        p, i = _sc_topk_sample(logits, cma, cmb, den, tail,
                               gpad[w * WROWS:(w + 1) * WROWS], w * WROWS)
        pp.append(p)
        ii.append(i)
    probs_pad = jnp.concatenate(pp, axis=0)
    ints_pad = jnp.concatenate(ii, axis=0)
    topk_probs = probs_pad[:, :TOPK]
    topk_indices = ints_pad[:, :TOPK]
    xcol = ints_pad[:, TOPK:TOPK + 1]
    return xcol, topk_probs, topk_indices
